# Initial kernel scaffold; baseline (speedup 1.0000x reference)
import jax
import jax.numpy as jnp
from jax.experimental import pallas as pl


def kernel(x, edge_index, W0, b0, W1, b1, W2, b2):
    raise NotImplementedError("write your pallas kernel here")


# speedup vs baseline: 4.8697x; 4.8697x over previous
import dataclasses
import functools

import jax
import jax.numpy as jnp
from jax import lax
from jax.experimental import pallas as pl
from jax.experimental.pallas import tpu as pltpu
from jax.experimental.pallas import tpu_sc as plsc

N = 10000
E = 320000
CHUNK = 80
ROWS = E // CHUNK
NC = 2
NS = 16
NW = NC * NS
DEG_W = 16
ZROWS = 200
WB = 1000


def _sc_mesh():
    return plsc.VectorSubcoreMesh(core_axis_name="c", subcore_axis_name="s")


def _sc_params():
    cp = pltpu.CompilerParams()
    if "needs_layout_passes" in pltpu.CompilerParams.__dataclass_fields__:
        cp = dataclasses.replace(cp, needs_layout_passes=False)
    return cp


HR = 80
IDXB = 2000


def _degrees(edge_flat):
    @functools.partial(
        pl.kernel,
        mesh=_sc_mesh(),
        out_type=jax.ShapeDtypeStruct((NC, HR, 128), jnp.float32),
        scratch_types=[
            pltpu.VMEM_SHARED((HR, 128), jnp.float32),
            pltpu.VMEM((IDXB,), jnp.int32),
            pltpu.VMEM((HR, 128), jnp.float32),
            pltpu.VMEM((1, HR), jnp.int32),
        ],
        compiler_params=_sc_params(),
    )
    def deg_kernel(e_hbm, deg_hbm, hist_sh, idx_v, hloc, riota):
        c = lax.axis_index("c")
        s = lax.axis_index("s")
        ones16 = jnp.ones((16,), jnp.float32)

        @pl.loop(0, HR)
        def _(i):
            for k in range(8):
                hloc[i, pl.ds(k * 16, 16)] = jnp.zeros((16,), jnp.float32)

        for k in range(HR // 16):
            riota[0, pl.ds(k * 16, 16)] = (
                lax.iota(jnp.int32, 16) + (k * 16))

        @pl.when(s == 0)
        def _():
            pltpu.sync_copy(hloc, hist_sh)

        per_w = E // NS

        @pl.loop(0, per_w // IDXB)
        def _(j):
            base = c * E + s * per_w + j * IDXB
            pltpu.sync_copy(e_hbm.at[pl.ds(base, IDXB)], idx_v)

            @pl.loop(0, IDXB // 16)
            def _(k):
                idx = idx_v[pl.ds(k * 16, 16)]
                row = lax.shift_right_logical(idx, 7)
                col = lax.bitwise_and(idx, 127)
                plsc.addupdate_scatter(hloc, [row, col], ones16)

        plsc.subcore_barrier()
        pltpu.sync_copy(hloc, hist_sh.at[riota.at[0]], add=True)
        plsc.subcore_barrier()

        @pl.when(s == 0)
        def _():
            pltpu.sync_copy(hist_sh, deg_hbm.at[c])

    return deg_kernel(edge_flat)


def _propagate(hs, src_flat, dst_flat, d):
    per_w = E // NW

    @functools.partial(
        pl.kernel,
        mesh=_sc_mesh(),
        out_type=jax.ShapeDtypeStruct((NC, N, d), jnp.float32),
        scratch_types=[
            pltpu.VMEM_SHARED((N, d), jnp.float32),
            pltpu.VMEM((2, CHUNK), jnp.int32),
            pltpu.VMEM((2, CHUNK), jnp.int32),
            pltpu.VMEM((2, CHUNK, d), jnp.float32),
            pltpu.VMEM((ZROWS, d), jnp.float32),
            pltpu.SemaphoreType.DMA,
        ],
    )
    def prop_kernel(hs_hbm, src_hbm, dst_hbm, out_hbm, accum, sidx, didx,
                    rows_v, zero_v, sem):
        c = lax.axis_index("c")
        s = lax.axis_index("s")
        w = c * NS + s

        @pl.loop(0, ZROWS)
        def _(i):
            for k in range(d // 16):
                zero_v[i, pl.ds(k * 16, 16)] = jnp.zeros((16,), jnp.float32)

        @pl.when(s < N // WB)
        def _():
            for k in range(WB // ZROWS):
                pltpu.sync_copy(
                    zero_v, accum.at[pl.ds(s * WB + k * ZROWS, ZROWS)])

        plsc.subcore_barrier()

        @pl.loop(0, per_w // CHUNK)
        def _(j):
            base = w * per_w + j * CHUNK
            esl = pl.ds(base, CHUNK)
            pltpu.sync_copy(src_hbm.at[esl], sidx.at[0])
            pltpu.sync_copy(dst_hbm.at[esl], didx.at[0])
            pltpu.async_copy(hs_hbm.at[sidx.at[0]], rows_v.at[0], sem).wait()
            pltpu.sync_copy(rows_v.at[0], accum.at[didx.at[0]], add=True)

        plsc.subcore_barrier()

        @pl.when(s < N // WB)
        def _():
            sl = pl.ds(s * WB, WB)
            pltpu.sync_copy(accum.at[sl], out_hbm.at[c].at[sl])

    return prop_kernel(hs, src_flat, dst_flat)


_TCR = 2000


def _tc_first(x, w, deg_out):
    def body(x_ref, w_ref, dego_ref, out_ref):
        ns = lax.rsqrt(jnp.maximum(dego_ref[...], 1.0))
        h = jnp.dot(x_ref[...], w_ref[...],
                    preferred_element_type=jnp.float32,
                    precision=lax.Precision.HIGHEST)
        out_ref[...] = h * ns

    return pl.pallas_call(
        body,
        grid=(N // _TCR,),
        in_specs=[
            pl.BlockSpec((_TCR, 128), lambda i: (i, 0)),
            pl.BlockSpec((128, 128), lambda i: (0, 0)),
            pl.BlockSpec((_TCR, 1), lambda i: (i, 0)),
        ],
        out_specs=pl.BlockSpec((_TCR, 128), lambda i: (i, 0)),
        out_shape=jax.ShapeDtypeStruct((N, 128), jnp.float32),
    )(x, w, deg_out)


def _tc_mid(p, deg_in, deg_out, b, w, d_next):
    def body(p_ref, din_ref, dego_ref, b_ref, w_ref, mid_ref, hs_ref):
        nd = lax.rsqrt(jnp.maximum(din_ref[...], 1.0))
        ns = lax.rsqrt(jnp.maximum(dego_ref[...], 1.0))
        mid = (p_ref[0] + p_ref[1]) * nd + b_ref[...]
        mid_ref[...] = mid
        h = jnp.dot(jnp.maximum(mid, 0.0), w_ref[...],
                    preferred_element_type=jnp.float32,
                    precision=lax.Precision.HIGHEST)
        hs_ref[...] = h * ns

    return pl.pallas_call(
        body,
        grid=(N // _TCR,),
        in_specs=[
            pl.BlockSpec((NC, _TCR, 128), lambda i: (0, i, 0)),
            pl.BlockSpec((_TCR, 1), lambda i: (i, 0)),
            pl.BlockSpec((_TCR, 1), lambda i: (i, 0)),
            pl.BlockSpec((1, 128), lambda i: (0, 0)),
            pl.BlockSpec((128, d_next), lambda i: (0, 0)),
        ],
        out_specs=[
            pl.BlockSpec((_TCR, 128), lambda i: (i, 0)),
            pl.BlockSpec((_TCR, d_next), lambda i: (i, 0)),
        ],
        out_shape=[
            jax.ShapeDtypeStruct((N, 128), jnp.float32),
            jax.ShapeDtypeStruct((N, d_next), jnp.float32),
        ],
    )(p, deg_in, deg_out, b, w)


def _tc_mid2(p, deg_in, deg_out, b):
    def body(p_ref, din_ref, dego_ref, b_ref, mid_ref, hs_ref):
        nd = lax.rsqrt(jnp.maximum(din_ref[...], 1.0))
        ns = lax.rsqrt(jnp.maximum(dego_ref[...], 1.0))
        mid = (p_ref[0] + p_ref[1]) * nd + b_ref[...]
        mid_ref[...] = mid
        hs_ref[...] = jnp.maximum(mid, 0.0) * ns

    return pl.pallas_call(
        body,
        grid=(N // _TCR,),
        in_specs=[
            pl.BlockSpec((NC, _TCR, 128), lambda i: (0, i, 0)),
            pl.BlockSpec((_TCR, 1), lambda i: (i, 0)),
            pl.BlockSpec((_TCR, 1), lambda i: (i, 0)),
            pl.BlockSpec((1, 128), lambda i: (0, 0)),
        ],
        out_specs=[
            pl.BlockSpec((_TCR, 128), lambda i: (i, 0)),
            pl.BlockSpec((_TCR, 128), lambda i: (i, 0)),
        ],
        out_shape=[
            jax.ShapeDtypeStruct((N, 128), jnp.float32),
            jax.ShapeDtypeStruct((N, 128), jnp.float32),
        ],
    )(p, deg_in, deg_out, b)


def _tc_final(p, deg_in, w, b):
    def body(p_ref, din_ref, w_ref, b_ref, out_ref):
        nd = lax.rsqrt(jnp.maximum(din_ref[...], 1.0))
        agg = (p_ref[0] + p_ref[1]) * nd
        out_ref[...] = jnp.dot(agg, w_ref[...],
                               preferred_element_type=jnp.float32,
                               precision=lax.Precision.HIGHEST) + b_ref[...]

    return pl.pallas_call(
        body,
        grid=(N // _TCR,),
        in_specs=[
            pl.BlockSpec((NC, _TCR, 128), lambda i: (0, i, 0)),
            pl.BlockSpec((_TCR, 1), lambda i: (i, 0)),
            pl.BlockSpec((128, 64), lambda i: (0, 0)),
            pl.BlockSpec((1, 64), lambda i: (0, 0)),
        ],
        out_specs=pl.BlockSpec((_TCR, 64), lambda i: (i, 0)),
        out_shape=jax.ShapeDtypeStruct((N, 64), jnp.float32),
    )(p, deg_in, w, b)


def kernel(x, edge_index, W0, b0, W1, b1, W2, b2):
    src_flat = edge_index[0]
    dst_flat = edge_index[1]

    deg_h = _degrees(edge_index.reshape(2 * E))
    dout = deg_h[0].reshape(-1)[:N].reshape(N, 1)
    din = deg_h[1].reshape(-1)[:N].reshape(N, 1)

    hs0 = _tc_first(x, W0, dout)
    p0 = _propagate(hs0, src_flat, dst_flat, 128)
    mid0, hs1 = _tc_mid(p0, din, dout, b0.reshape(1, 128), W1, 128)
    p1 = _propagate(hs1, src_flat, dst_flat, 128)
    mid1, hs2 = _tc_mid2(p1, din, dout, b1.reshape(1, 128))
    p2 = _propagate(hs2, src_flat, dst_flat, 128)
    logits = _tc_final(p2, din, W2, b2.reshape(1, 64))
    return logits, mid0, mid1


# speedup vs baseline: 12.3139x; 2.5287x over previous
import dataclasses
import functools

import jax
import jax.numpy as jnp
from jax import lax
from jax.experimental import pallas as pl
from jax.experimental.pallas import tpu as pltpu
from jax.experimental.pallas import tpu_sc as plsc

N = 10000
E = 320000
CHUNK = 40
NC = 2
NS = 16
NW = NC * NS
DEG_W = 16
ZROWS = 40
WB = 1000


def _sc_mesh():
    return plsc.VectorSubcoreMesh(core_axis_name="c", subcore_axis_name="s")


def _sc_params():
    cp = pltpu.CompilerParams()
    if "needs_layout_passes" in pltpu.CompilerParams.__dataclass_fields__:
        cp = dataclasses.replace(cp, needs_layout_passes=False)
    return cp


HR = 80
IDXB = 2000


def _degrees(edge_flat):
    @functools.partial(
        pl.kernel,
        mesh=_sc_mesh(),
        out_type=jax.ShapeDtypeStruct((NC, HR, 128), jnp.float32),
        scratch_types=[
            pltpu.VMEM_SHARED((HR, 128), jnp.float32),
            pltpu.VMEM((IDXB,), jnp.int32),
            pltpu.VMEM((HR, 128), jnp.float32),
            pltpu.VMEM((1, HR), jnp.int32),
        ],
        compiler_params=_sc_params(),
    )
    def deg_kernel(e_hbm, deg_hbm, hist_sh, idx_v, hloc, riota):
        c = lax.axis_index("c")
        s = lax.axis_index("s")
        ones16 = jnp.ones((16,), jnp.float32)

        @pl.loop(0, HR)
        def _(i):
            for k in range(8):
                hloc[i, pl.ds(k * 16, 16)] = jnp.zeros((16,), jnp.float32)

        for k in range(HR // 16):
            riota[0, pl.ds(k * 16, 16)] = (
                lax.iota(jnp.int32, 16) + (k * 16))

        @pl.when(s == 0)
        def _():
            pltpu.sync_copy(hloc, hist_sh)

        per_w = E // NS

        @pl.loop(0, per_w // IDXB)
        def _(j):
            base = c * E + s * per_w + j * IDXB
            pltpu.sync_copy(e_hbm.at[pl.ds(base, IDXB)], idx_v)

            @pl.loop(0, IDXB // 16)
            def _(k):
                idx = idx_v[pl.ds(k * 16, 16)]
                row = lax.shift_right_logical(idx, 7)
                col = lax.bitwise_and(idx, 127)
                plsc.addupdate_scatter(hloc, [row, col], ones16)

        plsc.subcore_barrier()
        pltpu.sync_copy(hloc, hist_sh.at[riota.at[0]], add=True)
        plsc.subcore_barrier()

        @pl.when(s == 0)
        def _():
            pltpu.sync_copy(hist_sh, deg_hbm.at[c])

    return deg_kernel(edge_flat)


NBUF = 5
IBLK = 2000
BCH = IBLK // CHUNK


def _propagate(hs, src_flat, dst_flat, d):
    per_w = E // NW
    nblk = per_w // IBLK

    @functools.partial(
        pl.kernel,
        mesh=_sc_mesh(),
        out_type=jax.ShapeDtypeStruct((NC, N, d), jnp.float32),
        scratch_types=[
            pltpu.VMEM_SHARED((N, d), jnp.float32),
            pltpu.VMEM((IBLK,), jnp.int32),
            pltpu.VMEM((IBLK,), jnp.int32),
            pltpu.VMEM((NBUF, CHUNK, d), jnp.float32),
            pltpu.VMEM((ZROWS, d), jnp.float32),
            pltpu.SemaphoreType.DMA((NBUF,)),
        ],
    )
    def prop_kernel(hs_hbm, src_hbm, dst_hbm, out_hbm, accum, sidx, didx,
                    rows_v, zero_v, sems):
        c = lax.axis_index("c")
        s = lax.axis_index("s")
        w = c * NS + s

        def fetch_idx(blk):
            esl = pl.ds(w * per_w + blk * IBLK, IBLK)
            pltpu.sync_copy(src_hbm.at[esl], sidx)
            pltpu.sync_copy(dst_hbm.at[esl], didx)

        def prime():
            for b in range(NBUF):
                pltpu.async_copy(
                    hs_hbm.at[sidx.at[pl.ds(b * CHUNK, CHUNK)]],
                    rows_v.at[b], sems.at[b])

        fetch_idx(0)
        prime()

        @pl.loop(0, ZROWS)
        def _(i):
            for k in range(d // 16):
                zero_v[i, pl.ds(k * 16, 16)] = jnp.zeros((16,), jnp.float32)

        @pl.when(s < N // WB)
        def _():
            for k in range(WB // ZROWS):
                pltpu.sync_copy(
                    zero_v, accum.at[pl.ds(s * WB + k * ZROWS, ZROWS)])

        plsc.subcore_barrier()

        @pl.loop(0, nblk)
        def _(blk):
            @pl.when(blk > 0)
            def _():
                fetch_idx(blk)
                prime()

            @pl.loop(0, BCH, step=NBUF)
            def _(g):
                for b in range(NBUF):
                    j = g + b
                    pltpu.make_async_copy(
                        hs_hbm.at[sidx.at[pl.ds(j * CHUNK, CHUNK)]],
                        rows_v.at[b], sems.at[b]).wait()
                    pltpu.sync_copy(
                        rows_v.at[b],
                        accum.at[didx.at[pl.ds(j * CHUNK, CHUNK)]], add=True)
                    nj = j + NBUF

                    @pl.when(nj < BCH)
                    def _():
                        pltpu.async_copy(
                            hs_hbm.at[sidx.at[pl.ds(nj * CHUNK, CHUNK)]],
                            rows_v.at[b], sems.at[b])

        plsc.subcore_barrier()

        @pl.when(s < N // WB)
        def _():
            sl = pl.ds(s * WB, WB)
            pltpu.sync_copy(accum.at[sl], out_hbm.at[c].at[sl])

    return prop_kernel(hs, src_flat, dst_flat)


_TCR = 2000


def _tc_first(x, w, deg_out):
    def body(x_ref, w_ref, dego_ref, out_ref):
        ns = lax.rsqrt(jnp.maximum(dego_ref[...], 1.0))
        h = jnp.dot(x_ref[...], w_ref[...],
                    preferred_element_type=jnp.float32,
                    precision=lax.Precision.HIGHEST)
        out_ref[...] = h * ns

    return pl.pallas_call(
        body,
        grid=(N // _TCR,),
        in_specs=[
            pl.BlockSpec((_TCR, 128), lambda i: (i, 0)),
            pl.BlockSpec((128, 128), lambda i: (0, 0)),
            pl.BlockSpec((_TCR, 1), lambda i: (i, 0)),
        ],
        out_specs=pl.BlockSpec((_TCR, 128), lambda i: (i, 0)),
        out_shape=jax.ShapeDtypeStruct((N, 128), jnp.float32),
    )(x, w, deg_out)


def _tc_mid(p, deg_in, deg_out, b, w, d_next):
    def body(p_ref, din_ref, dego_ref, b_ref, w_ref, mid_ref, hs_ref):
        nd = lax.rsqrt(jnp.maximum(din_ref[...], 1.0))
        ns = lax.rsqrt(jnp.maximum(dego_ref[...], 1.0))
        mid = (p_ref[0] + p_ref[1]) * nd + b_ref[...]
        mid_ref[...] = mid
        h = jnp.dot(jnp.maximum(mid, 0.0), w_ref[...],
                    preferred_element_type=jnp.float32,
                    precision=lax.Precision.HIGHEST)
        hs_ref[...] = h * ns

    return pl.pallas_call(
        body,
        grid=(N // _TCR,),
        in_specs=[
            pl.BlockSpec((NC, _TCR, 128), lambda i: (0, i, 0)),
            pl.BlockSpec((_TCR, 1), lambda i: (i, 0)),
            pl.BlockSpec((_TCR, 1), lambda i: (i, 0)),
            pl.BlockSpec((1, 128), lambda i: (0, 0)),
            pl.BlockSpec((128, d_next), lambda i: (0, 0)),
        ],
        out_specs=[
            pl.BlockSpec((_TCR, 128), lambda i: (i, 0)),
            pl.BlockSpec((_TCR, d_next), lambda i: (i, 0)),
        ],
        out_shape=[
            jax.ShapeDtypeStruct((N, 128), jnp.float32),
            jax.ShapeDtypeStruct((N, d_next), jnp.float32),
        ],
    )(p, deg_in, deg_out, b, w)


def _tc_mid2(p, deg_in, deg_out, b):
    def body(p_ref, din_ref, dego_ref, b_ref, mid_ref, hs_ref):
        nd = lax.rsqrt(jnp.maximum(din_ref[...], 1.0))
        ns = lax.rsqrt(jnp.maximum(dego_ref[...], 1.0))
        mid = (p_ref[0] + p_ref[1]) * nd + b_ref[...]
        mid_ref[...] = mid
        hs_ref[...] = jnp.maximum(mid, 0.0) * ns

    return pl.pallas_call(
        body,
        grid=(N // _TCR,),
        in_specs=[
            pl.BlockSpec((NC, _TCR, 128), lambda i: (0, i, 0)),
            pl.BlockSpec((_TCR, 1), lambda i: (i, 0)),
            pl.BlockSpec((_TCR, 1), lambda i: (i, 0)),
            pl.BlockSpec((1, 128), lambda i: (0, 0)),
        ],
        out_specs=[
            pl.BlockSpec((_TCR, 128), lambda i: (i, 0)),
            pl.BlockSpec((_TCR, 128), lambda i: (i, 0)),
        ],
        out_shape=[
            jax.ShapeDtypeStruct((N, 128), jnp.float32),
            jax.ShapeDtypeStruct((N, 128), jnp.float32),
        ],
    )(p, deg_in, deg_out, b)


def _tc_final(p, deg_in, w, b):
    def body(p_ref, din_ref, w_ref, b_ref, out_ref):
        nd = lax.rsqrt(jnp.maximum(din_ref[...], 1.0))
        agg = (p_ref[0] + p_ref[1]) * nd
        out_ref[...] = jnp.dot(agg, w_ref[...],
                               preferred_element_type=jnp.float32,
                               precision=lax.Precision.HIGHEST) + b_ref[...]

    return pl.pallas_call(
        body,
        grid=(N // _TCR,),
        in_specs=[
            pl.BlockSpec((NC, _TCR, 128), lambda i: (0, i, 0)),
            pl.BlockSpec((_TCR, 1), lambda i: (i, 0)),
            pl.BlockSpec((128, 64), lambda i: (0, 0)),
            pl.BlockSpec((1, 64), lambda i: (0, 0)),
        ],
        out_specs=pl.BlockSpec((_TCR, 64), lambda i: (i, 0)),
        out_shape=jax.ShapeDtypeStruct((N, 64), jnp.float32),
    )(p, deg_in, w, b)


def kernel(x, edge_index, W0, b0, W1, b1, W2, b2):
    src_flat = edge_index[0]
    dst_flat = edge_index[1]

    deg_h = _degrees(edge_index.reshape(2 * E))
    dout = deg_h[0].reshape(-1)[:N].reshape(N, 1)
    din = deg_h[1].reshape(-1)[:N].reshape(N, 1)

    hs0 = _tc_first(x, W0, dout)
    p0 = _propagate(hs0, src_flat, dst_flat, 128)
    mid0, hs1 = _tc_mid(p0, din, dout, b0.reshape(1, 128), W1, 128)
    p1 = _propagate(hs1, src_flat, dst_flat, 128)
    mid1, hs2 = _tc_mid2(p1, din, dout, b1.reshape(1, 128))
    p2 = _propagate(hs2, src_flat, dst_flat, 128)
    logits = _tc_final(p2, din, W2, b2.reshape(1, 64))
    return logits, mid0, mid1
